# baseline (speedup 1.0000x reference)
import jax
import jax.numpy as jnp
from jax.experimental import pallas as pl
from jax.experimental.pallas import tpu as pltpu

_CONF = 0.596
_IOU = 0.45
_NDET = 300
_NPAD = 304
_NCAND = 20000


def _score_kernel(x_ref, s_ref):
    x = x_ref[...]
    obj = x[:, 4:5]
    cls_s = x[:, 5:85] * obj
    conf = jnp.max(cls_s, axis=1, keepdims=True)
    valid = (obj > _CONF) & (conf > _CONF)
    s_ref[...] = jnp.where(valid, conf, -1.0)


def _selnms_kernel(s_in_ref, preds_ref, selT_ref, s_ref):
    s_ref[...] = s_in_ref[...]
    ri = jax.lax.broadcasted_iota(jnp.int32, (8, 2500), 0)
    ci = jax.lax.broadcasted_iota(jnp.int32, (8, 2500), 1)
    flat = ri * 2500 + ci
    ci304 = jax.lax.broadcasted_iota(jnp.int32, (1, _NPAD), 1)
    iota80 = jax.lax.broadcasted_iota(jnp.int32, (1, 80), 1)
    zrow = jnp.zeros((1, _NPAD), jnp.float32)

    def sel_body(i, carry):
        x1v, y1v, x2v, y2v, clsv, scv = carry
        s = s_ref[...]
        m = jnp.max(s)
        idx = jnp.min(jnp.where(s == m, flat, _NCAND))
        s_ref[...] = jnp.where(flat == idx, -2.0, s)
        prow = preds_ref[pl.ds(idx, 1), :]
        cx = jnp.sum(prow[:, 0:1])
        cy = jnp.sum(prow[:, 1:2])
        w = jnp.sum(prow[:, 2:3])
        h = jnp.sum(prow[:, 3:4])
        obj = jnp.sum(prow[:, 4:5])
        cls_s = prow[:, 5:85] * obj
        cmax = jnp.max(cls_s)
        cls = jnp.min(jnp.where(cls_s == cmax, iota80, 80)).astype(jnp.float32)
        hit = ci304 == i
        x1v = jnp.where(hit, cx - w * 0.5, x1v)
        y1v = jnp.where(hit, cy - h * 0.5, y1v)
        x2v = jnp.where(hit, cx + w * 0.5, x2v)
        y2v = jnp.where(hit, cy + h * 0.5, y2v)
        clsv = jnp.where(hit, cls, clsv)
        scv = jnp.where(hit, m, scv)
        return x1v, y1v, x2v, y2v, clsv, scv

    x1v, y1v, x2v, y2v, clsv, scv = jax.lax.fori_loop(
        0, _NDET, sel_body, (zrow, zrow, zrow, zrow, zrow, zrow))

    off = clsv * 4096.0
    nx1 = x1v + off
    ny1 = y1v + off
    nx2 = x2v + off
    ny2 = y2v + off
    area = (nx2 - nx1) * (ny2 - ny1)

    def nms_body(i, keep):
        m = (ci304 == i).astype(jnp.float32)
        bx1 = jnp.sum(nx1 * m)
        by1 = jnp.sum(ny1 * m)
        bx2 = jnp.sum(nx2 * m)
        by2 = jnp.sum(ny2 * m)
        barea = jnp.sum(area * m)
        ki = jnp.sum(keep * m)
        iw = jnp.clip(jnp.minimum(bx2, nx2) - jnp.maximum(bx1, nx1), 0.0, None)
        ih = jnp.clip(jnp.minimum(by2, ny2) - jnp.maximum(by1, ny1), 0.0, None)
        inter = iw * ih
        iou = inter / (barea + area - inter + 1e-9)
        sup = (iou > _IOU) & (ci304 > i) & (ki > 0.5)
        return jnp.where(sup, 0.0, keep)

    keep = jax.lax.fori_loop(0, _NDET, nms_body,
                             (scv > 0.0).astype(jnp.float32))

    selT_ref[0:1, :] = x1v
    selT_ref[1:2, :] = y1v
    selT_ref[2:3, :] = x2v
    selT_ref[3:4, :] = y2v
    selT_ref[4:5, :] = clsv
    selT_ref[5:6, :] = scv
    selT_ref[6:7, :] = keep
    selT_ref[7:8, :] = zrow


def _roi_kernel(sel_ref, f1_ref, f2_ref, f3_ref, f4_ref, out_ref):
    levels = (
        (f1_ref, 1.0 / 8, 96, 96, 1, 0),
        (f2_ref, 1.0 / 16, 48, 48, 2, 1),
        (f3_ref, 1.0 / 32, 24, 24, 4, 3),
        (f4_ref, 1.0 / 64, 12, 12, 8, 7),
    )
    for j in range(8):
        bx1 = jnp.sum(sel_ref[j:j + 1, 0:1])
        by1 = jnp.sum(sel_ref[j:j + 1, 1:2])
        bx2 = jnp.sum(sel_ref[j:j + 1, 2:3])
        by2 = jnp.sum(sel_ref[j:j + 1, 3:4])
        for fref, s, H, W, p, qoff in levels:
            x1 = bx1 * s
            y1 = by1 * s
            x2 = bx2 * s
            y2 = by2 * s
            xA = x1 + 0.25 * (x2 - x1) - 0.5
            xB = x1 + 0.75 * (x2 - x1) - 0.5
            yA = y1 + 0.25 * (y2 - y1) - 0.5
            yB = y1 + 0.75 * (y2 - y1) - 0.5
            acc = jnp.zeros((p, 128), jnp.float32)
            for yy, xx in ((yA, xA), (yA, xB), (yB, xA), (yB, xB)):
                y = jnp.clip(yy, 0.0, H - 1.0)
                x = jnp.clip(xx, 0.0, W - 1.0)
                y0f = jnp.floor(y)
                x0f = jnp.floor(x)
                y0 = y0f.astype(jnp.int32)
                x0 = x0f.astype(jnp.int32)
                y1i = jnp.minimum(y0 + 1, H - 1)
                x1i = jnp.minimum(x0 + 1, W - 1)
                ly = y - y0f
                lx = x - x0f
                r00 = fref[pl.ds((y0 * W + x0) * p, p), :]
                r01 = fref[pl.ds((y0 * W + x1i) * p, p), :]
                r10 = fref[pl.ds((y1i * W + x0) * p, p), :]
                r11 = fref[pl.ds((y1i * W + x1i) * p, p), :]
                acc = acc + (r00 * ((1.0 - ly) * (1.0 - lx))
                             + r01 * ((1.0 - ly) * lx)
                             + r10 * (ly * (1.0 - lx))
                             + r11 * (ly * lx))
            out_ref[16 * j + qoff:16 * j + qoff + p, :] = acc * 0.25
        out_ref[16 * j + 15:16 * j + 16, :] = jnp.zeros((1, 128), jnp.float32)


def _mlp_kernel(f_ref, sel_ref, wa_ref, ba_ref, wb_ref, bb_ref, out_ref):
    f = f_ref[...]
    h = jnp.dot(f, wa_ref[...], preferred_element_type=jnp.float32) + ba_ref[...]
    h = jnp.where(h >= 0, h, 0.01 * h)
    h = jnp.dot(h, wb_ref[...], preferred_element_type=jnp.float32) + bb_ref[...]
    h = jnp.where(h >= 0, h, 0.01 * h)
    keep = sel_ref[:, 6:7]
    nb = sel_ref[:, 0:4] * (1.0 / 96.0)
    out_ref[...] = jnp.concatenate([nb, h], axis=1) * keep


def _pack(feat, hw):
    c = feat.shape[0]
    p = c // 128
    return feat.reshape(p, 128, hw).transpose(2, 0, 1).reshape(hw * p, 128)


def kernel(preds, feat1, feat2, feat3, feat4, Wa, ba, Wb, bb):
    x = preds[0]
    masked = pl.pallas_call(
        _score_kernel,
        grid=(10,),
        in_specs=[pl.BlockSpec((2000, 85), lambda i: (i, 0))],
        out_specs=pl.BlockSpec((2000, 1), lambda i: (i, 0)),
        out_shape=jax.ShapeDtypeStruct((_NCAND, 1), jnp.float32),
    )(x)
    s2d = masked.reshape(8, 2500)
    selT = pl.pallas_call(
        _selnms_kernel,
        out_shape=jax.ShapeDtypeStruct((8, _NPAD), jnp.float32),
        scratch_shapes=[pltpu.VMEM((8, 2500), jnp.float32)],
    )(s2d, x)
    sel8 = selT.T
    f1p = _pack(feat1[0], 96 * 96)
    f2p = _pack(feat2[0], 48 * 48)
    f3p = _pack(feat3[0], 24 * 24)
    f4p = _pack(feat4[0], 12 * 12)
    froi = pl.pallas_call(
        _roi_kernel,
        grid=(_NPAD // 8,),
        in_specs=[
            pl.BlockSpec((8, 8), lambda k: (k, 0)),
            pl.BlockSpec((9216, 128), lambda k: (0, 0)),
            pl.BlockSpec((4608, 128), lambda k: (0, 0)),
            pl.BlockSpec((2304, 128), lambda k: (0, 0)),
            pl.BlockSpec((1152, 128), lambda k: (0, 0)),
        ],
        out_specs=pl.BlockSpec((128, 128), lambda k: (k, 0)),
        out_shape=jax.ShapeDtypeStruct((_NPAD * 16, 128), jnp.float32),
    )(sel8, f1p, f2p, f3p, f4p)
    f2048 = froi.reshape(_NPAD, 2048)
    wa_pad = jnp.concatenate([Wa, jnp.zeros((128, 64), jnp.float32)], axis=0)
    out = pl.pallas_call(
        _mlp_kernel,
        out_shape=jax.ShapeDtypeStruct((_NPAD, 68), jnp.float32),
    )(f2048, sel8, wa_pad, ba.reshape(1, 64), Wb, bb.reshape(1, 64))
    return out[:_NDET]


# speedup vs baseline: 1.3051x; 1.3051x over previous
import jax
import jax.numpy as jnp
from jax.experimental import pallas as pl
from jax.experimental.pallas import tpu as pltpu

_CONF = 0.596
_IOU = 0.45
_NDET = 300
_NPAD = 304
_NCAND = 20000


def _score_kernel(x_ref, meta_ref):
    x = x_ref[...]
    obj = x[:, 4:5]
    cls_s = x[:, 5:85] * obj
    conf = jnp.max(cls_s, axis=1, keepdims=True)
    li = jax.lax.broadcasted_iota(jnp.int32, cls_s.shape, 1)
    cls = jnp.min(jnp.where(cls_s == conf, li, 80), axis=1, keepdims=True)
    valid = (obj > _CONF) & (conf > _CONF)
    masked = jnp.where(valid, conf, -1.0)
    cxcy = x[:, 0:2]
    wh = x[:, 2:4]
    meta_ref[...] = jnp.concatenate(
        [cxcy - wh * 0.5, cxcy + wh * 0.5, cls.astype(jnp.float32), masked],
        axis=1,
    )


def _topk_kernel(s_in_ref, meta_ref, sel_ref, s_ref):
    s_ref[...] = s_in_ref[...]
    ri = jax.lax.broadcasted_iota(jnp.int32, (8, 2500), 0)
    ci = jax.lax.broadcasted_iota(jnp.int32, (8, 2500), 1)
    flat = ri * 2500 + ci
    rsel = jax.lax.broadcasted_iota(jnp.int32, (_NPAD, 6), 0)
    sel_ref[...] = jnp.zeros((_NPAD, 6), jnp.float32)

    def body(i, carry):
        s = s_ref[...]
        m = jnp.max(s)
        idx = jnp.min(jnp.where(s == m, flat, _NCAND))
        s_ref[...] = jnp.where(flat == idx, -2.0, s)
        row = meta_ref[pl.ds(idx, 1), :]
        sel_ref[...] = jnp.where(rsel == i,
                                 jnp.broadcast_to(row, (_NPAD, 6)),
                                 sel_ref[...])
        return carry

    jax.lax.fori_loop(0, _NDET, body, 0)


def _nms_kernel(selT_ref, keep_ref):
    off = selT_ref[4:5, :] * 4096.0
    x1 = selT_ref[0:1, :] + off
    y1 = selT_ref[1:2, :] + off
    x2 = selT_ref[2:3, :] + off
    y2 = selT_ref[3:4, :] + off
    sc = selT_ref[5:6, :]
    area = (x2 - x1) * (y2 - y1)
    ji = jax.lax.broadcasted_iota(jnp.int32, (1, _NPAD), 1)
    keep_ref[...] = (sc > 0.0).astype(jnp.float32)

    def body(i, carry):
        keep = keep_ref[...]
        m = (ji == i).astype(jnp.float32)
        bx1 = jnp.sum(x1 * m)
        by1 = jnp.sum(y1 * m)
        bx2 = jnp.sum(x2 * m)
        by2 = jnp.sum(y2 * m)
        barea = jnp.sum(area * m)
        ki = jnp.sum(keep * m)
        iw = jnp.clip(jnp.minimum(bx2, x2) - jnp.maximum(bx1, x1), 0.0, None)
        ih = jnp.clip(jnp.minimum(by2, y2) - jnp.maximum(by1, y1), 0.0, None)
        inter = iw * ih
        iou = inter / (barea + area - inter + 1e-9)
        sup = (iou > _IOU) & (ji > i) & (ki > 0.5)
        keep_ref[...] = jnp.where(sup, 0.0, keep)
        return carry

    jax.lax.fori_loop(0, _NDET, body, 0)


def _roi_kernel(sel_ref, f1_ref, f2_ref, f3_ref, f4_ref, out_ref):
    levels = (
        (f1_ref, 1.0 / 8, 96, 96, 1, 0),
        (f2_ref, 1.0 / 16, 48, 48, 2, 1),
        (f3_ref, 1.0 / 32, 24, 24, 4, 3),
        (f4_ref, 1.0 / 64, 12, 12, 8, 7),
    )
    for j in range(8):
        bx1 = jnp.sum(sel_ref[j:j + 1, 0:1])
        by1 = jnp.sum(sel_ref[j:j + 1, 1:2])
        bx2 = jnp.sum(sel_ref[j:j + 1, 2:3])
        by2 = jnp.sum(sel_ref[j:j + 1, 3:4])
        for fref, s, H, W, p, qoff in levels:
            x1 = bx1 * s
            y1 = by1 * s
            x2 = bx2 * s
            y2 = by2 * s
            xA = x1 + 0.25 * (x2 - x1) - 0.5
            xB = x1 + 0.75 * (x2 - x1) - 0.5
            yA = y1 + 0.25 * (y2 - y1) - 0.5
            yB = y1 + 0.75 * (y2 - y1) - 0.5
            acc = jnp.zeros((p, 128), jnp.float32)
            for yy, xx in ((yA, xA), (yA, xB), (yB, xA), (yB, xB)):
                y = jnp.clip(yy, 0.0, H - 1.0)
                x = jnp.clip(xx, 0.0, W - 1.0)
                y0f = jnp.floor(y)
                x0f = jnp.floor(x)
                y0 = y0f.astype(jnp.int32)
                x0 = x0f.astype(jnp.int32)
                y1i = jnp.minimum(y0 + 1, H - 1)
                x1i = jnp.minimum(x0 + 1, W - 1)
                ly = y - y0f
                lx = x - x0f
                r00 = fref[pl.ds((y0 * W + x0) * p, p), :]
                r01 = fref[pl.ds((y0 * W + x1i) * p, p), :]
                r10 = fref[pl.ds((y1i * W + x0) * p, p), :]
                r11 = fref[pl.ds((y1i * W + x1i) * p, p), :]
                acc = acc + (r00 * ((1.0 - ly) * (1.0 - lx))
                             + r01 * ((1.0 - ly) * lx)
                             + r10 * (ly * (1.0 - lx))
                             + r11 * (ly * lx))
            out_ref[16 * j + qoff:16 * j + qoff + p, :] = acc * 0.25
        out_ref[16 * j + 15:16 * j + 16, :] = jnp.zeros((1, 128), jnp.float32)


def _mlp_kernel(f_ref, sel_ref, keep_ref, wa_ref, ba_ref, wb_ref, bb_ref,
                out_ref):
    f = f_ref[...]
    h = jnp.dot(f, wa_ref[...], preferred_element_type=jnp.float32) + ba_ref[...]
    h = jnp.where(h >= 0, h, 0.01 * h)
    h = jnp.dot(h, wb_ref[...], preferred_element_type=jnp.float32) + bb_ref[...]
    h = jnp.where(h >= 0, h, 0.01 * h)
    keep = keep_ref[...]
    nb = sel_ref[:, 0:4] * (1.0 / 96.0)
    out_ref[...] = jnp.concatenate([nb, h], axis=1) * keep


def _pack(feat, hw):
    c = feat.shape[0]
    p = c // 128
    return feat.reshape(p, 128, hw).transpose(2, 0, 1).reshape(hw * p, 128)


def kernel(preds, feat1, feat2, feat3, feat4, Wa, ba, Wb, bb):
    x = preds[0]
    meta = pl.pallas_call(
        _score_kernel,
        grid=(10,),
        in_specs=[pl.BlockSpec((2000, 85), lambda i: (i, 0))],
        out_specs=pl.BlockSpec((2000, 6), lambda i: (i, 0)),
        out_shape=jax.ShapeDtypeStruct((_NCAND, 6), jnp.float32),
    )(x)
    s2d = meta[:, 5].reshape(8, 2500)
    sel = pl.pallas_call(
        _topk_kernel,
        out_shape=jax.ShapeDtypeStruct((_NPAD, 6), jnp.float32),
        scratch_shapes=[pltpu.VMEM((8, 2500), jnp.float32)],
    )(s2d, meta)
    keep = pl.pallas_call(
        _nms_kernel,
        out_shape=jax.ShapeDtypeStruct((1, _NPAD), jnp.float32),
    )(sel.T)
    f1p = _pack(feat1[0], 96 * 96)
    f2p = _pack(feat2[0], 48 * 48)
    f3p = _pack(feat3[0], 24 * 24)
    f4p = _pack(feat4[0], 12 * 12)
    froi = pl.pallas_call(
        _roi_kernel,
        grid=(_NPAD // 8,),
        in_specs=[
            pl.BlockSpec((8, 6), lambda k: (k, 0)),
            pl.BlockSpec((9216, 128), lambda k: (0, 0)),
            pl.BlockSpec((4608, 128), lambda k: (0, 0)),
            pl.BlockSpec((2304, 128), lambda k: (0, 0)),
            pl.BlockSpec((1152, 128), lambda k: (0, 0)),
        ],
        out_specs=pl.BlockSpec((128, 128), lambda k: (k, 0)),
        out_shape=jax.ShapeDtypeStruct((_NPAD * 16, 128), jnp.float32),
    )(sel, f1p, f2p, f3p, f4p)
    f2048 = froi.reshape(_NPAD, 2048)
    wa_pad = jnp.concatenate([Wa, jnp.zeros((128, 64), jnp.float32)], axis=0)
    out = pl.pallas_call(
        _mlp_kernel,
        out_shape=jax.ShapeDtypeStruct((_NPAD, 68), jnp.float32),
    )(f2048, sel, keep.T, wa_pad, ba.reshape(1, 64), Wb, bb.reshape(1, 64))
    return out[:_NDET]


# speedup vs baseline: 1.4063x; 1.0775x over previous
import jax
import jax.numpy as jnp
from jax.experimental import pallas as pl
from jax.experimental.pallas import tpu as pltpu

_CONF = 0.596
_IOU = 0.45
_NDET = 300
_NPAD = 304
_NCAND = 20000


def _score_kernel(x_ref, s_ref):
    x = x_ref[...]
    obj = x[:, 4:5]
    cls_s = x[:, 5:85] * obj
    conf = jnp.max(cls_s, axis=1, keepdims=True)
    valid = (obj > _CONF) & (conf > _CONF)
    s_ref[...] = jnp.where(valid, conf, -1.0)


def _topk_kernel(s_in_ref, preds_ref, selraw_ref, s_ref):
    s_ref[...] = s_in_ref[...]
    ri = jax.lax.broadcasted_iota(jnp.int32, (8, 2500), 0)
    ci = jax.lax.broadcasted_iota(jnp.int32, (8, 2500), 1)
    flat = ri * 2500 + ci
    rbuf = jax.lax.broadcasted_iota(jnp.int32, (8, 96), 0)
    zpad = jnp.zeros((1, 10), jnp.float32)

    def outer(k, carry):
        buf = jnp.zeros((8, 96), jnp.float32)
        for j in range(8):
            s = s_ref[...]
            m = jnp.max(s)
            idx = jnp.min(jnp.where(s == m, flat, _NCAND))
            s_ref[...] = jnp.where(flat == idx, -2.0, s)
            prow = preds_ref[pl.ds(idx, 1), :]
            row = jnp.concatenate(
                [prow, jnp.full((1, 1), m, jnp.float32), zpad], axis=1)
            buf = jnp.where(rbuf == j, jnp.broadcast_to(row, (8, 96)), buf)
        selraw_ref[pl.ds(k * 8, 8), :] = buf
        return carry

    jax.lax.fori_loop(0, _NPAD // 8, outer, 0)


def _postsel_kernel(r_ref, sel_ref):
    r = r_ref[...]
    obj = r[:, 4:5]
    cls_s = r[:, 5:85] * obj
    cmax = jnp.max(cls_s, axis=1, keepdims=True)
    li = jax.lax.broadcasted_iota(jnp.int32, cls_s.shape, 1)
    cls = jnp.min(jnp.where(cls_s == cmax, li, 80), axis=1, keepdims=True)
    cxcy = r[:, 0:2]
    wh = r[:, 2:4]
    sel_ref[...] = jnp.concatenate(
        [cxcy - wh * 0.5, cxcy + wh * 0.5, cls.astype(jnp.float32),
         r[:, 85:86]],
        axis=1,
    )


def _nms_kernel(selT_ref, selS_ref, keep_ref):
    off = selT_ref[4:5, :] * 4096.0
    x1 = selT_ref[0:1, :] + off
    y1 = selT_ref[1:2, :] + off
    x2 = selT_ref[2:3, :] + off
    y2 = selT_ref[3:4, :] + off
    sc = selT_ref[5:6, :]
    area = (x2 - x1) * (y2 - y1)
    ji = jax.lax.broadcasted_iota(jnp.int32, (1, _NPAD), 1)
    keep_ref[...] = (sc > 0.0).astype(jnp.float32)

    def body(i, carry):
        keep = keep_ref[...]
        c = selS_ref[i, 4] * 4096.0
        bx1 = selS_ref[i, 0] + c
        by1 = selS_ref[i, 1] + c
        bx2 = selS_ref[i, 2] + c
        by2 = selS_ref[i, 3] + c
        barea = (bx2 - bx1) * (by2 - by1)
        ki = jnp.sum(keep * (ji == i).astype(jnp.float32))
        iw = jnp.clip(jnp.minimum(bx2, x2) - jnp.maximum(bx1, x1), 0.0, None)
        ih = jnp.clip(jnp.minimum(by2, y2) - jnp.maximum(by1, y1), 0.0, None)
        inter = iw * ih
        iou = inter / (barea + area - inter + 1e-9)
        sup = (iou > _IOU) & (ji > i) & (ki > 0.5)
        keep_ref[...] = jnp.where(sup, 0.0, keep)
        return carry

    jax.lax.fori_loop(0, _NDET, body, 0)


def _roi_kernel(sel_ref, f1_ref, f2_ref, f3_ref, f4_ref, out_ref):
    levels = (
        (f1_ref, 1.0 / 8, 96, 96, 1, 0),
        (f2_ref, 1.0 / 16, 48, 48, 2, 1),
        (f3_ref, 1.0 / 32, 24, 24, 4, 3),
        (f4_ref, 1.0 / 64, 12, 12, 8, 7),
    )
    for j in range(8):
        bx1 = jnp.sum(sel_ref[j:j + 1, 0:1])
        by1 = jnp.sum(sel_ref[j:j + 1, 1:2])
        bx2 = jnp.sum(sel_ref[j:j + 1, 2:3])
        by2 = jnp.sum(sel_ref[j:j + 1, 3:4])
        for fref, s, H, W, p, qoff in levels:
            x1 = bx1 * s
            y1 = by1 * s
            x2 = bx2 * s
            y2 = by2 * s
            xA = x1 + 0.25 * (x2 - x1) - 0.5
            xB = x1 + 0.75 * (x2 - x1) - 0.5
            yA = y1 + 0.25 * (y2 - y1) - 0.5
            yB = y1 + 0.75 * (y2 - y1) - 0.5
            acc = jnp.zeros((p, 128), jnp.float32)
            for yy, xx in ((yA, xA), (yA, xB), (yB, xA), (yB, xB)):
                y = jnp.clip(yy, 0.0, H - 1.0)
                x = jnp.clip(xx, 0.0, W - 1.0)
                y0f = jnp.floor(y)
                x0f = jnp.floor(x)
                y0 = y0f.astype(jnp.int32)
                x0 = x0f.astype(jnp.int32)
                y1i = jnp.minimum(y0 + 1, H - 1)
                x1i = jnp.minimum(x0 + 1, W - 1)
                ly = y - y0f
                lx = x - x0f
                r00 = fref[pl.ds((y0 * W + x0) * p, p), :]
                r01 = fref[pl.ds((y0 * W + x1i) * p, p), :]
                r10 = fref[pl.ds((y1i * W + x0) * p, p), :]
                r11 = fref[pl.ds((y1i * W + x1i) * p, p), :]
                acc = acc + (r00 * ((1.0 - ly) * (1.0 - lx))
                             + r01 * ((1.0 - ly) * lx)
                             + r10 * (ly * (1.0 - lx))
                             + r11 * (ly * lx))
            out_ref[16 * j + qoff:16 * j + qoff + p, :] = acc * 0.25
        out_ref[16 * j + 15:16 * j + 16, :] = jnp.zeros((1, 128), jnp.float32)


def _mlp_kernel(f_ref, sel_ref, keep_ref, wa_ref, ba_ref, wb_ref, bb_ref,
                out_ref):
    f = f_ref[...]
    h = jnp.dot(f, wa_ref[...], preferred_element_type=jnp.float32) + ba_ref[...]
    h = jnp.where(h >= 0, h, 0.01 * h)
    h = jnp.dot(h, wb_ref[...], preferred_element_type=jnp.float32) + bb_ref[...]
    h = jnp.where(h >= 0, h, 0.01 * h)
    keep = keep_ref[...]
    nb = sel_ref[:, 0:4] * (1.0 / 96.0)
    out_ref[...] = jnp.concatenate([nb, h], axis=1) * keep


def _pack(feat, hw):
    c = feat.shape[0]
    p = c // 128
    return feat.reshape(p, 128, hw).transpose(2, 0, 1).reshape(hw * p, 128)


def kernel(preds, feat1, feat2, feat3, feat4, Wa, ba, Wb, bb):
    x = preds[0]
    masked = pl.pallas_call(
        _score_kernel,
        grid=(10,),
        in_specs=[pl.BlockSpec((2000, 85), lambda i: (i, 0))],
        out_specs=pl.BlockSpec((2000, 1), lambda i: (i, 0)),
        out_shape=jax.ShapeDtypeStruct((_NCAND, 1), jnp.float32),
    )(x)
    s2d = masked.reshape(8, 2500)
    selraw = pl.pallas_call(
        _topk_kernel,
        out_shape=jax.ShapeDtypeStruct((_NPAD, 96), jnp.float32),
        scratch_shapes=[pltpu.VMEM((8, 2500), jnp.float32)],
    )(s2d, x)
    sel = pl.pallas_call(
        _postsel_kernel,
        out_shape=jax.ShapeDtypeStruct((_NPAD, 6), jnp.float32),
    )(selraw)
    keep = pl.pallas_call(
        _nms_kernel,
        in_specs=[
            pl.BlockSpec((6, _NPAD), lambda: (0, 0)),
            pl.BlockSpec(memory_space=pltpu.SMEM),
        ],
        out_shape=jax.ShapeDtypeStruct((1, _NPAD), jnp.float32),
    )(sel.T, sel)
    f1p = _pack(feat1[0], 96 * 96)
    f2p = _pack(feat2[0], 48 * 48)
    f3p = _pack(feat3[0], 24 * 24)
    f4p = _pack(feat4[0], 12 * 12)
    froi = pl.pallas_call(
        _roi_kernel,
        grid=(_NPAD // 8,),
        in_specs=[
            pl.BlockSpec((8, 6), lambda k: (k, 0)),
            pl.BlockSpec((9216, 128), lambda k: (0, 0)),
            pl.BlockSpec((4608, 128), lambda k: (0, 0)),
            pl.BlockSpec((2304, 128), lambda k: (0, 0)),
            pl.BlockSpec((1152, 128), lambda k: (0, 0)),
        ],
        out_specs=pl.BlockSpec((128, 128), lambda k: (k, 0)),
        out_shape=jax.ShapeDtypeStruct((_NPAD * 16, 128), jnp.float32),
    )(sel, f1p, f2p, f3p, f4p)
    f2048 = froi.reshape(_NPAD, 2048)
    wa_pad = jnp.concatenate([Wa, jnp.zeros((128, 64), jnp.float32)], axis=0)
    out = pl.pallas_call(
        _mlp_kernel,
        out_shape=jax.ShapeDtypeStruct((_NPAD, 68), jnp.float32),
    )(f2048, sel, keep.T, wa_pad, ba.reshape(1, 64), Wb, bb.reshape(1, 64))
    return out[:_NDET]


# speedup vs baseline: 1.4128x; 1.0046x over previous
import jax
import jax.numpy as jnp
from jax.experimental import pallas as pl
from jax.experimental.pallas import tpu as pltpu

_CONF = 0.596
_IOU = 0.45
_NDET = 300
_NPAD = 304
_NCAND = 20000


def _score_kernel(x_ref, s_ref):
    x = x_ref[...]
    obj = x[:, 4:5]
    cls_s = x[:, 5:85] * obj
    conf = jnp.max(cls_s, axis=1, keepdims=True)
    valid = (obj > _CONF) & (conf > _CONF)
    s_ref[...] = jnp.where(valid, conf, -1.0)


def _topk_kernel(s_in_ref, preds_ref, selraw_ref):
    ri = jax.lax.broadcasted_iota(jnp.int32, (8, 2500), 0)
    ci = jax.lax.broadcasted_iota(jnp.int32, (8, 2500), 1)
    flat = ri * 2500 + ci
    rbuf = jax.lax.broadcasted_iota(jnp.int32, (8, 96), 0)
    zpad = jnp.zeros((1, 10), jnp.float32)

    def outer(k, s):
        buf = jnp.zeros((8, 96), jnp.float32)
        for j in range(8):
            m = jnp.max(s)
            idx = jnp.min(jnp.where(s == m, flat, _NCAND))
            s = jnp.where(flat == idx, -2.0, s)
            prow = preds_ref[pl.ds(idx, 1), :]
            row = jnp.concatenate(
                [prow, jnp.full((1, 1), m, jnp.float32), zpad], axis=1)
            buf = jnp.where(rbuf == j, jnp.broadcast_to(row, (8, 96)), buf)
        selraw_ref[pl.ds(k * 8, 8), :] = buf
        return s

    jax.lax.fori_loop(0, _NPAD // 8, outer, s_in_ref[...])


def _postsel_kernel(r_ref, sel_ref):
    r = r_ref[...]
    obj = r[:, 4:5]
    cls_s = r[:, 5:85] * obj
    cmax = jnp.max(cls_s, axis=1, keepdims=True)
    li = jax.lax.broadcasted_iota(jnp.int32, cls_s.shape, 1)
    cls = jnp.min(jnp.where(cls_s == cmax, li, 80), axis=1, keepdims=True)
    cxcy = r[:, 0:2]
    wh = r[:, 2:4]
    sel_ref[...] = jnp.concatenate(
        [cxcy - wh * 0.5, cxcy + wh * 0.5, cls.astype(jnp.float32),
         r[:, 85:86]],
        axis=1,
    )


def _nms_kernel(selT_ref, selS_ref, keep_ref):
    off = selT_ref[4:5, :] * 4096.0
    x1 = selT_ref[0:1, :] + off
    y1 = selT_ref[1:2, :] + off
    x2 = selT_ref[2:3, :] + off
    y2 = selT_ref[3:4, :] + off
    sc = selT_ref[5:6, :]
    area = (x2 - x1) * (y2 - y1)
    ji = jax.lax.broadcasted_iota(jnp.int32, (1, _NPAD), 1)

    def body(i, keep):
        c = selS_ref[i, 4] * 4096.0
        bx1 = selS_ref[i, 0] + c
        by1 = selS_ref[i, 1] + c
        bx2 = selS_ref[i, 2] + c
        by2 = selS_ref[i, 3] + c
        barea = (bx2 - bx1) * (by2 - by1)
        ki = jnp.sum(keep * (ji == i).astype(jnp.float32))
        iw = jnp.clip(jnp.minimum(bx2, x2) - jnp.maximum(bx1, x1), 0.0, None)
        ih = jnp.clip(jnp.minimum(by2, y2) - jnp.maximum(by1, y1), 0.0, None)
        inter = iw * ih
        iou = inter / (barea + area - inter + 1e-9)
        sup = (iou > _IOU) & (ji > i) & (ki > 0.5)
        return jnp.where(sup, 0.0, keep)

    keep_ref[...] = jax.lax.fori_loop(0, _NDET, body,
                                      (sc > 0.0).astype(jnp.float32))


def _roi_kernel(sel_ref, f1_ref, f2_ref, f3_ref, f4_ref, out_ref):
    levels = (
        (f1_ref, 1.0 / 8, 96, 96, 1, 0),
        (f2_ref, 1.0 / 16, 48, 48, 2, 1),
        (f3_ref, 1.0 / 32, 24, 24, 4, 3),
        (f4_ref, 1.0 / 64, 12, 12, 8, 7),
    )
    for j in range(8):
        bx1 = jnp.sum(sel_ref[j:j + 1, 0:1])
        by1 = jnp.sum(sel_ref[j:j + 1, 1:2])
        bx2 = jnp.sum(sel_ref[j:j + 1, 2:3])
        by2 = jnp.sum(sel_ref[j:j + 1, 3:4])
        for fref, s, H, W, p, qoff in levels:
            x1 = bx1 * s
            y1 = by1 * s
            x2 = bx2 * s
            y2 = by2 * s
            xA = x1 + 0.25 * (x2 - x1) - 0.5
            xB = x1 + 0.75 * (x2 - x1) - 0.5
            yA = y1 + 0.25 * (y2 - y1) - 0.5
            yB = y1 + 0.75 * (y2 - y1) - 0.5
            acc = jnp.zeros((p, 128), jnp.float32)
            for yy, xx in ((yA, xA), (yA, xB), (yB, xA), (yB, xB)):
                y = jnp.clip(yy, 0.0, H - 1.0)
                x = jnp.clip(xx, 0.0, W - 1.0)
                y0f = jnp.floor(y)
                x0f = jnp.floor(x)
                y0 = y0f.astype(jnp.int32)
                x0 = x0f.astype(jnp.int32)
                y1i = jnp.minimum(y0 + 1, H - 1)
                x1i = jnp.minimum(x0 + 1, W - 1)
                ly = y - y0f
                lx = x - x0f
                r00 = fref[pl.ds((y0 * W + x0) * p, p), :]
                r01 = fref[pl.ds((y0 * W + x1i) * p, p), :]
                r10 = fref[pl.ds((y1i * W + x0) * p, p), :]
                r11 = fref[pl.ds((y1i * W + x1i) * p, p), :]
                acc = acc + (r00 * ((1.0 - ly) * (1.0 - lx))
                             + r01 * ((1.0 - ly) * lx)
                             + r10 * (ly * (1.0 - lx))
                             + r11 * (ly * lx))
            out_ref[16 * j + qoff:16 * j + qoff + p, :] = acc * 0.25
        out_ref[16 * j + 15:16 * j + 16, :] = jnp.zeros((1, 128), jnp.float32)


def _mlp_kernel(f_ref, sel_ref, keep_ref, wa_ref, ba_ref, wb_ref, bb_ref,
                out_ref):
    f = f_ref[...]
    h = jnp.dot(f, wa_ref[...], preferred_element_type=jnp.float32) + ba_ref[...]
    h = jnp.where(h >= 0, h, 0.01 * h)
    h = jnp.dot(h, wb_ref[...], preferred_element_type=jnp.float32) + bb_ref[...]
    h = jnp.where(h >= 0, h, 0.01 * h)
    keep = keep_ref[...]
    nb = sel_ref[:, 0:4] * (1.0 / 96.0)
    out_ref[...] = jnp.concatenate([nb, h], axis=1) * keep


def _pack(feat, hw):
    c = feat.shape[0]
    p = c // 128
    return feat.reshape(p, 128, hw).transpose(2, 0, 1).reshape(hw * p, 128)


def kernel(preds, feat1, feat2, feat3, feat4, Wa, ba, Wb, bb):
    x = preds[0]
    masked = pl.pallas_call(
        _score_kernel,
        grid=(10,),
        in_specs=[pl.BlockSpec((2000, 85), lambda i: (i, 0))],
        out_specs=pl.BlockSpec((2000, 1), lambda i: (i, 0)),
        out_shape=jax.ShapeDtypeStruct((_NCAND, 1), jnp.float32),
    )(x)
    s2d = masked.reshape(8, 2500)
    selraw = pl.pallas_call(
        _topk_kernel,
        out_shape=jax.ShapeDtypeStruct((_NPAD, 96), jnp.float32),
    )(s2d, x)
    sel = pl.pallas_call(
        _postsel_kernel,
        out_shape=jax.ShapeDtypeStruct((_NPAD, 6), jnp.float32),
    )(selraw)
    keep = pl.pallas_call(
        _nms_kernel,
        in_specs=[
            pl.BlockSpec((6, _NPAD), lambda: (0, 0)),
            pl.BlockSpec(memory_space=pltpu.SMEM),
        ],
        out_shape=jax.ShapeDtypeStruct((1, _NPAD), jnp.float32),
    )(sel.T, sel)
    f1p = _pack(feat1[0], 96 * 96)
    f2p = _pack(feat2[0], 48 * 48)
    f3p = _pack(feat3[0], 24 * 24)
    f4p = _pack(feat4[0], 12 * 12)
    froi = pl.pallas_call(
        _roi_kernel,
        grid=(_NPAD // 8,),
        in_specs=[
            pl.BlockSpec((8, 6), lambda k: (k, 0)),
            pl.BlockSpec((9216, 128), lambda k: (0, 0)),
            pl.BlockSpec((4608, 128), lambda k: (0, 0)),
            pl.BlockSpec((2304, 128), lambda k: (0, 0)),
            pl.BlockSpec((1152, 128), lambda k: (0, 0)),
        ],
        out_specs=pl.BlockSpec((128, 128), lambda k: (k, 0)),
        out_shape=jax.ShapeDtypeStruct((_NPAD * 16, 128), jnp.float32),
    )(sel, f1p, f2p, f3p, f4p)
    f2048 = froi.reshape(_NPAD, 2048)
    wa_pad = jnp.concatenate([Wa, jnp.zeros((128, 64), jnp.float32)], axis=0)
    out = pl.pallas_call(
        _mlp_kernel,
        out_shape=jax.ShapeDtypeStruct((_NPAD, 68), jnp.float32),
    )(f2048, sel, keep.T, wa_pad, ba.reshape(1, 64), Wb, bb.reshape(1, 64))
    return out[:_NDET]
